# baseline (speedup 1.0000x reference)
import jax
import jax.numpy as jnp
from jax.experimental import pallas as pl
from jax.experimental.pallas import tpu as pltpu

NUM_CATEGORIES = 1000


def _onehot_body(inp_hbm, out_ref, vbuf, sem):
    f = pl.program_id(0)

    @pl.when(f == 0)
    def _():
        pltpu.make_async_copy(inp_hbm, vbuf, sem).start()
        pltpu.make_async_copy(inp_hbm, vbuf, sem).wait()

    v = vbuf[pl.ds(f, 1), :]
    iota = jax.lax.broadcasted_iota(
        jnp.int32, (1, NUM_CATEGORIES, v.shape[1]), 1
    )
    out_ref[...] = (iota == v[:, None, :]).astype(jnp.float32)


def kernel(inputs):
    batch, nfeat = inputs.shape
    vt = inputs.astype(jnp.int32).T
    out_t = pl.pallas_call(
        _onehot_body,
        grid=(nfeat,),
        in_specs=[pl.BlockSpec(memory_space=pl.ANY)],
        out_specs=pl.BlockSpec((1, NUM_CATEGORIES, batch), lambda f: (f, 0, 0)),
        out_shape=jax.ShapeDtypeStruct((nfeat, NUM_CATEGORIES, batch), jnp.float32),
        scratch_shapes=[
            pltpu.VMEM((nfeat, batch), jnp.int32),
            pltpu.SemaphoreType.DMA,
        ],
    )(vt)
    return jnp.transpose(out_t, (2, 0, 1))
